# baseline (speedup 1.0000x reference)
import jax
import jax.numpy as jnp
import numpy as np
from jax.experimental import pallas as pl

B, S, H, NH = 1, 2048, 1024, 16
DH = H // NH

TS = 256
TQ = 512
TK = 512

NEG_INF = -1e30
BF = jnp.bfloat16


def _ln(t, w, b, eps=1e-6):
    m = jnp.mean(t, axis=-1, keepdims=True)
    v = jnp.mean((t - m) ** 2, axis=-1, keepdims=True)
    return (t - m) * jax.lax.rsqrt(v + eps) * w + b


def _dot_t(a, w):
    return jax.lax.dot_general(a, w, (((1,), (1,)), ((), ())),
                               preferred_element_type=jnp.float32)


def _qkv_kernel(x_ref, wq_ref, wk_ref, wv_ref, b_ref, lnw_ref, lnb_ref, out_ref):
    h = _ln(x_ref[...], lnw_ref[...], lnb_ref[...]).astype(BF)
    b = b_ref[...]
    out_ref[:, :H] = (_dot_t(h, wq_ref[...]) + b[:, :H]).astype(BF)
    out_ref[:, H:2 * H] = (_dot_t(h, wk_ref[...]) + b[:, H:2 * H]).astype(BF)
    out_ref[:, 2 * H:] = (_dot_t(h, wv_ref[...]) + b[:, 2 * H:]).astype(BF)


def _attn_kernel(q_ref, k_ref, v_ref, out_ref):
    qscale = jnp.float32(np.log2(np.e) / np.sqrt(DH))
    k = k_ref[0]
    ve = jnp.concatenate(
        [v_ref[0], jnp.ones((S, DH), BF)], axis=1)
    tri = jax.lax.broadcasted_iota(jnp.int32, (TQ, TQ), 0) >= \
        jax.lax.broadcasted_iota(jnp.int32, (TQ, TQ), 1)

    for i in range(S // TQ):
        lo = i * TQ
        qi = (q_ref[0, lo:lo + TQ, :].astype(jnp.float32) * qscale).astype(BF)
        sD = jax.lax.dot_general(qi, k[lo:lo + TQ], (((1,), (1,)), ((), ())),
                                 preferred_element_type=jnp.float32).astype(BF)
        sD = jnp.where(tri, sD, jnp.bfloat16(NEG_INF))
        mD = jnp.max(sD, axis=1, keepdims=True)
        if i > 0:
            sL = jax.lax.dot_general(qi, k[:lo], (((1,), (1,)), ((), ())),
                                     preferred_element_type=jnp.float32).astype(BF)
            m = jnp.maximum(mD, jnp.max(sL, axis=1, keepdims=True))
            pL = jnp.exp2(sL - m)
            pD = jnp.exp2(sD - m)
            acc = jnp.dot(pL, ve[:lo], preferred_element_type=jnp.float32)
            acc = acc + jnp.dot(pD, ve[lo:lo + TQ],
                                preferred_element_type=jnp.float32)
        else:
            pD = jnp.exp2(sD - mD)
            acc = jnp.dot(pD, ve[lo:lo + TQ],
                          preferred_element_type=jnp.float32)
        l = acc[:, DH:DH + 1]
        out_ref[0, lo:lo + TQ, :] = (acc[:, :DH] / l).astype(BF)


def _mlp_kernel(a_ref, x_ref, wo_ref, bo_ref, wg_ref, bg_ref, wu_ref, bu_ref,
                wd_ref, bd_ref, lnw_ref, lnb_ref, out_ref):
    at = jnp.concatenate([a_ref[h] for h in range(NH)], axis=1)
    x2 = _dot_t(at, wo_ref[...]) + bo_ref[...] + x_ref[...]
    h = _ln(x2, lnw_ref[...], lnb_ref[...]).astype(BF)
    g = _dot_t(h, wg_ref[...]) + bg_ref[...]
    u = _dot_t(h, wu_ref[...]) + bu_ref[...]
    mlp = ((g * jax.nn.sigmoid(g)) * u).astype(BF)
    out_ref[...] = _dot_t(mlp, wd_ref[...]) + bd_ref[...] + x2


def kernel(x, Wq, bq, Wk, bk, Wv, bv, Wo, bo, Wg, bg, Wu, bu, Wd, bd,
           ln1_w, ln1_b, ln2_w, ln2_b):
    xs = x.reshape(S, H)
    bqkv = jnp.concatenate([bq, bk, bv]).reshape(1, 3 * H)

    full = lambda shape: pl.BlockSpec(shape, lambda i: (0,) * len(shape))

    qkv = pl.pallas_call(
        _qkv_kernel,
        grid=(S // TS,),
        in_specs=[
            pl.BlockSpec((TS, H), lambda i: (i, 0)),
            full((H, H)), full((H, H)), full((H, H)),
            full((1, 3 * H)), full((1, H)), full((1, H)),
        ],
        out_specs=pl.BlockSpec((TS, 3 * H), lambda i: (i, 0)),
        out_shape=jax.ShapeDtypeStruct((S, 3 * H), BF),
    )(xs, Wq.astype(BF), Wk.astype(BF), Wv.astype(BF), bqkv,
      ln1_w.reshape(1, H), ln1_b.reshape(1, H))

    qkv_h = qkv.reshape(S, 3 * NH, DH).transpose(1, 0, 2)

    attn = pl.pallas_call(
        _attn_kernel,
        grid=(NH,),
        in_specs=[
            pl.BlockSpec((1, S, DH), lambda h: (h, 0, 0)),
            pl.BlockSpec((1, S, DH), lambda h: (NH + h, 0, 0)),
            pl.BlockSpec((1, S, DH), lambda h: (2 * NH + h, 0, 0)),
        ],
        out_specs=pl.BlockSpec((1, S, DH), lambda h: (h, 0, 0)),
        out_shape=jax.ShapeDtypeStruct((NH, S, DH), BF),
    )(qkv_h, qkv_h, qkv_h)

    out = pl.pallas_call(
        _mlp_kernel,
        grid=(S // TS,),
        in_specs=[
            pl.BlockSpec((NH, TS, DH), lambda i: (0, i, 0)),
            pl.BlockSpec((TS, H), lambda i: (i, 0)),
            full((H, H)), full((1, H)),
            full((H, H)), full((1, H)),
            full((H, H)), full((1, H)),
            full((H, H)), full((1, H)),
            full((1, H)), full((1, H)),
        ],
        out_specs=pl.BlockSpec((TS, H), lambda i: (i, 0)),
        out_shape=jax.ShapeDtypeStruct((S, H), jnp.float32),
    )(attn, xs, Wo.astype(BF), bo.reshape(1, H), Wg.astype(BF), bg.reshape(1, H),
      Wu.astype(BF), bu.reshape(1, H), Wd.astype(BF), bd.reshape(1, H),
      ln2_w.reshape(1, H), ln2_b.reshape(1, H))

    return out.reshape(B, S, H)


# speedup vs baseline: 1.0690x; 1.0690x over previous
import jax
import jax.numpy as jnp
import numpy as np
from jax.experimental import pallas as pl

B, S, H, NH = 1, 2048, 1024, 16
DH = H // NH

TS = 256
TQ = 512
TK = 512
HPP = 2

NEG_INF = -1e30
BF = jnp.bfloat16


def _ln(t, w, b, eps=1e-6):
    m = jnp.mean(t, axis=-1, keepdims=True)
    v = jnp.mean((t - m) ** 2, axis=-1, keepdims=True)
    return (t - m) * jax.lax.rsqrt(v + eps) * w + b


def _dot_t(a, w):
    return jax.lax.dot_general(a, w, (((1,), (1,)), ((), ())),
                               preferred_element_type=jnp.float32)


def _qkv_kernel(x_ref, wq_ref, wk_ref, wv_ref, b_ref, lnw_ref, lnb_ref, out_ref):
    h = _ln(x_ref[...], lnw_ref[...], lnb_ref[...]).astype(BF)
    b = b_ref[...]
    out_ref[:, :H] = (_dot_t(h, wq_ref[...]) + b[:, :H]).astype(BF)
    out_ref[:, H:2 * H] = (_dot_t(h, wk_ref[...]) + b[:, H:2 * H]).astype(BF)
    out_ref[:, 2 * H:] = (_dot_t(h, wv_ref[...]) + b[:, 2 * H:]).astype(BF)


def _attn_kernel(q_ref, k_ref, v_ref, out_ref):
    qscale = jnp.float32(np.log2(np.e) / np.sqrt(DH))
    tri = jax.lax.broadcasted_iota(jnp.int32, (TQ, TQ), 0) >= \
        jax.lax.broadcasted_iota(jnp.int32, (TQ, TQ), 1)

    for hh, i in [(a, b) for b in range(S // TQ) for a in range(HPP)]:
        k = k_ref[hh]
        ve = jnp.concatenate(
            [v_ref[hh], jnp.ones((S, DH), BF)], axis=1)
        lo = i * TQ
        qi = (q_ref[hh, lo:lo + TQ, :].astype(jnp.float32) * qscale).astype(BF)
        sD = jax.lax.dot_general(qi, k[lo:lo + TQ], (((1,), (1,)), ((), ())),
                                 preferred_element_type=jnp.float32).astype(BF)
        sD = jnp.where(tri, sD, jnp.bfloat16(NEG_INF))
        mD = jnp.max(sD, axis=1, keepdims=True)
        if i > 0:
            sL = jax.lax.dot_general(qi, k[:lo], (((1,), (1,)), ((), ())),
                                     preferred_element_type=jnp.float32).astype(BF)
            m = jnp.maximum(mD, jnp.max(sL, axis=1, keepdims=True))
            pL = jnp.exp2(sL - m)
            pD = jnp.exp2(sD - m)
            acc = jnp.dot(pL, ve[:lo], preferred_element_type=jnp.float32)
            acc = acc + jnp.dot(pD, ve[lo:lo + TQ],
                                preferred_element_type=jnp.float32)
        else:
            pD = jnp.exp2(sD - mD)
            acc = jnp.dot(pD, ve[lo:lo + TQ],
                          preferred_element_type=jnp.float32)
        l = acc[:, DH:DH + 1]
        out_ref[hh, lo:lo + TQ, :] = (acc[:, :DH] / l).astype(BF)


def _mlp_kernel(a_ref, x_ref, wo_ref, bo_ref, wg_ref, bg_ref, wu_ref, bu_ref,
                wd_ref, bd_ref, lnw_ref, lnb_ref, out_ref):
    at = jnp.concatenate([a_ref[h] for h in range(NH)], axis=1)
    x2 = _dot_t(at, wo_ref[...]) + bo_ref[...] + x_ref[...]
    h = _ln(x2, lnw_ref[...], lnb_ref[...]).astype(BF)
    g = _dot_t(h, wg_ref[...]) + bg_ref[...]
    u = _dot_t(h, wu_ref[...]) + bu_ref[...]
    mlp = ((g * jax.nn.sigmoid(g)) * u).astype(BF)
    out_ref[...] = _dot_t(mlp, wd_ref[...]) + bd_ref[...] + x2


def kernel(x, Wq, bq, Wk, bk, Wv, bv, Wo, bo, Wg, bg, Wu, bu, Wd, bd,
           ln1_w, ln1_b, ln2_w, ln2_b):
    xs = x.reshape(S, H)
    bqkv = jnp.concatenate([bq, bk, bv]).reshape(1, 3 * H)

    full = lambda shape: pl.BlockSpec(shape, lambda i: (0,) * len(shape))

    qkv = pl.pallas_call(
        _qkv_kernel,
        grid=(S // TS,),
        in_specs=[
            pl.BlockSpec((TS, H), lambda i: (i, 0)),
            full((H, H)), full((H, H)), full((H, H)),
            full((1, 3 * H)), full((1, H)), full((1, H)),
        ],
        out_specs=pl.BlockSpec((TS, 3 * H), lambda i: (i, 0)),
        out_shape=jax.ShapeDtypeStruct((S, 3 * H), BF),
    )(xs, Wq.astype(BF), Wk.astype(BF), Wv.astype(BF), bqkv,
      ln1_w.reshape(1, H), ln1_b.reshape(1, H))

    qkv_h = qkv.reshape(S, 3 * NH, DH).transpose(1, 0, 2)

    attn = pl.pallas_call(
        _attn_kernel,
        grid=(NH // HPP,),
        in_specs=[
            pl.BlockSpec((HPP, S, DH), lambda h: (h, 0, 0)),
            pl.BlockSpec((HPP, S, DH), lambda h: (NH // HPP + h, 0, 0)),
            pl.BlockSpec((HPP, S, DH), lambda h: (2 * NH // HPP + h, 0, 0)),
        ],
        out_specs=pl.BlockSpec((HPP, S, DH), lambda h: (h, 0, 0)),
        out_shape=jax.ShapeDtypeStruct((NH, S, DH), BF),
    )(qkv_h, qkv_h, qkv_h)

    out = pl.pallas_call(
        _mlp_kernel,
        grid=(S // TS,),
        in_specs=[
            pl.BlockSpec((NH, TS, DH), lambda i: (0, i, 0)),
            pl.BlockSpec((TS, H), lambda i: (i, 0)),
            full((H, H)), full((1, H)),
            full((H, H)), full((1, H)),
            full((H, H)), full((1, H)),
            full((H, H)), full((1, H)),
            full((1, H)), full((1, H)),
        ],
        out_specs=pl.BlockSpec((TS, H), lambda i: (i, 0)),
        out_shape=jax.ShapeDtypeStruct((S, H), jnp.float32),
    )(attn, xs, Wo.astype(BF), bo.reshape(1, H), Wg.astype(BF), bg.reshape(1, H),
      Wu.astype(BF), bu.reshape(1, H), Wd.astype(BF), bd.reshape(1, H),
      ln2_w.reshape(1, H), ln2_b.reshape(1, H))

    return out.reshape(B, S, H)


# speedup vs baseline: 1.1040x; 1.0328x over previous
import jax
import jax.numpy as jnp
import numpy as np
from jax.experimental import pallas as pl

B, S, H, NH = 1, 2048, 1024, 16
DH = H // NH

TS = 256
TQ = 512
TK = 512
HPP = 4

NEG_INF = -1e30
BF = jnp.bfloat16


def _ln(t, w, b, eps=1e-6):
    m = jnp.mean(t, axis=-1, keepdims=True)
    v = jnp.mean((t - m) ** 2, axis=-1, keepdims=True)
    return (t - m) * jax.lax.rsqrt(v + eps) * w + b


def _dot_t(a, w):
    return jax.lax.dot_general(a, w, (((1,), (1,)), ((), ())),
                               preferred_element_type=jnp.float32)


def _qkv_kernel(x_ref, wq_ref, wk_ref, wv_ref, b_ref, lnw_ref, lnb_ref, out_ref):
    h = _ln(x_ref[...], lnw_ref[...], lnb_ref[...]).astype(BF)
    b = b_ref[...]
    out_ref[:, :H] = (_dot_t(h, wq_ref[...]) + b[:, :H]).astype(BF)
    out_ref[:, H:2 * H] = (_dot_t(h, wk_ref[...]) + b[:, H:2 * H]).astype(BF)
    out_ref[:, 2 * H:] = (_dot_t(h, wv_ref[...]) + b[:, 2 * H:]).astype(BF)


def _attn_kernel(q_ref, k_ref, v_ref, out_ref):
    qscale = jnp.float32(np.log2(np.e) / np.sqrt(DH))
    tri = jax.lax.broadcasted_iota(jnp.int32, (TQ, TQ), 0) >= \
        jax.lax.broadcasted_iota(jnp.int32, (TQ, TQ), 1)

    for hh, i in [(a, b) for b in range(S // TQ) for a in range(HPP)]:
        k = k_ref[hh]
        ve = jnp.concatenate(
            [v_ref[hh], jnp.ones((S, DH), BF)], axis=1)
        lo = i * TQ
        qi = (q_ref[hh, lo:lo + TQ, :].astype(jnp.float32) * qscale).astype(BF)
        sD = jax.lax.dot_general(qi, k[lo:lo + TQ], (((1,), (1,)), ((), ())),
                                 preferred_element_type=jnp.float32).astype(BF)
        sD = jnp.where(tri, sD, jnp.bfloat16(NEG_INF))
        mD = jnp.max(sD, axis=1, keepdims=True)
        if i > 0:
            sL = jax.lax.dot_general(qi, k[:lo], (((1,), (1,)), ((), ())),
                                     preferred_element_type=jnp.float32).astype(BF)
            m = jnp.maximum(mD, jnp.max(sL, axis=1, keepdims=True))
            pL = jnp.exp2(sL - m)
            pD = jnp.exp2(sD - m)
            acc = jnp.dot(pL, ve[:lo], preferred_element_type=jnp.float32)
            acc = acc + jnp.dot(pD, ve[lo:lo + TQ],
                                preferred_element_type=jnp.float32)
        else:
            pD = jnp.exp2(sD - mD)
            acc = jnp.dot(pD, ve[lo:lo + TQ],
                          preferred_element_type=jnp.float32)
        l = acc[:, DH:DH + 1]
        out_ref[hh, lo:lo + TQ, :] = (acc[:, :DH] / l).astype(BF)


def _mlp_kernel(a_ref, x_ref, wo_ref, bo_ref, wg_ref, bg_ref, wu_ref, bu_ref,
                wd_ref, bd_ref, lnw_ref, lnb_ref, out_ref):
    at = jnp.concatenate([a_ref[h] for h in range(NH)], axis=1)
    x2 = _dot_t(at, wo_ref[...]) + bo_ref[...] + x_ref[...]
    h = _ln(x2, lnw_ref[...], lnb_ref[...]).astype(BF)
    g = _dot_t(h, wg_ref[...]) + bg_ref[...]
    u = _dot_t(h, wu_ref[...]) + bu_ref[...]
    mlp = ((g * jax.nn.sigmoid(g)) * u).astype(BF)
    out_ref[...] = _dot_t(mlp, wd_ref[...]) + bd_ref[...] + x2


def kernel(x, Wq, bq, Wk, bk, Wv, bv, Wo, bo, Wg, bg, Wu, bu, Wd, bd,
           ln1_w, ln1_b, ln2_w, ln2_b):
    xs = x.reshape(S, H)
    bqkv = jnp.concatenate([bq, bk, bv]).reshape(1, 3 * H)

    full = lambda shape: pl.BlockSpec(shape, lambda i: (0,) * len(shape))

    qkv = pl.pallas_call(
        _qkv_kernel,
        grid=(S // TS,),
        in_specs=[
            pl.BlockSpec((TS, H), lambda i: (i, 0)),
            full((H, H)), full((H, H)), full((H, H)),
            full((1, 3 * H)), full((1, H)), full((1, H)),
        ],
        out_specs=pl.BlockSpec((TS, 3 * H), lambda i: (i, 0)),
        out_shape=jax.ShapeDtypeStruct((S, 3 * H), BF),
    )(xs, Wq.astype(BF), Wk.astype(BF), Wv.astype(BF), bqkv,
      ln1_w.reshape(1, H), ln1_b.reshape(1, H))

    qkv_h = qkv.reshape(S, 3 * NH, DH).transpose(1, 0, 2)

    attn = pl.pallas_call(
        _attn_kernel,
        grid=(NH // HPP,),
        in_specs=[
            pl.BlockSpec((HPP, S, DH), lambda h: (h, 0, 0)),
            pl.BlockSpec((HPP, S, DH), lambda h: (NH // HPP + h, 0, 0)),
            pl.BlockSpec((HPP, S, DH), lambda h: (2 * NH // HPP + h, 0, 0)),
        ],
        out_specs=pl.BlockSpec((HPP, S, DH), lambda h: (h, 0, 0)),
        out_shape=jax.ShapeDtypeStruct((NH, S, DH), BF),
    )(qkv_h, qkv_h, qkv_h)

    out = pl.pallas_call(
        _mlp_kernel,
        grid=(S // TS,),
        in_specs=[
            pl.BlockSpec((NH, TS, DH), lambda i: (0, i, 0)),
            pl.BlockSpec((TS, H), lambda i: (i, 0)),
            full((H, H)), full((1, H)),
            full((H, H)), full((1, H)),
            full((H, H)), full((1, H)),
            full((H, H)), full((1, H)),
            full((1, H)), full((1, H)),
        ],
        out_specs=pl.BlockSpec((TS, H), lambda i: (i, 0)),
        out_shape=jax.ShapeDtypeStruct((S, H), jnp.float32),
    )(attn, xs, Wo.astype(BF), bo.reshape(1, H), Wg.astype(BF), bg.reshape(1, H),
      Wu.astype(BF), bu.reshape(1, H), Wd.astype(BF), bd.reshape(1, H),
      ln2_w.reshape(1, H), ln2_b.reshape(1, H))

    return out.reshape(B, S, H)


# speedup vs baseline: 1.1167x; 1.0115x over previous
import jax
import jax.numpy as jnp
import numpy as np
from jax.experimental import pallas as pl

B, S, H, NH = 1, 2048, 1024, 16
DH = H // NH

TS = 256
TQ = 512
TK = 512
HPP = 8

NEG_INF = -1e30
BF = jnp.bfloat16


def _ln(t, w, b, eps=1e-6):
    m = jnp.mean(t, axis=-1, keepdims=True)
    v = jnp.mean((t - m) ** 2, axis=-1, keepdims=True)
    return (t - m) * jax.lax.rsqrt(v + eps) * w + b


def _dot_t(a, w):
    return jax.lax.dot_general(a, w, (((1,), (1,)), ((), ())),
                               preferred_element_type=jnp.float32)


def _qkv_kernel(x_ref, wq_ref, wk_ref, wv_ref, b_ref, lnw_ref, lnb_ref, out_ref):
    h = _ln(x_ref[...], lnw_ref[...], lnb_ref[...]).astype(BF)
    b = b_ref[...]
    out_ref[:, :H] = (_dot_t(h, wq_ref[...]) + b[:, :H]).astype(BF)
    out_ref[:, H:2 * H] = (_dot_t(h, wk_ref[...]) + b[:, H:2 * H]).astype(BF)
    out_ref[:, 2 * H:] = (_dot_t(h, wv_ref[...]) + b[:, 2 * H:]).astype(BF)


def _attn_kernel(q_ref, k_ref, v_ref, out_ref):
    qscale = jnp.float32(np.log2(np.e) / np.sqrt(DH))
    tri = jax.lax.broadcasted_iota(jnp.int32, (TQ, TQ), 0) >= \
        jax.lax.broadcasted_iota(jnp.int32, (TQ, TQ), 1)

    for hh, i in [(a, b) for b in range(S // TQ) for a in range(HPP)]:
        k = k_ref[hh]
        ve = jnp.concatenate(
            [v_ref[hh], jnp.ones((S, DH), BF)], axis=1)
        lo = i * TQ
        qi = (q_ref[hh, lo:lo + TQ, :].astype(jnp.float32) * qscale).astype(BF)
        sD = jax.lax.dot_general(qi, k[lo:lo + TQ], (((1,), (1,)), ((), ())),
                                 preferred_element_type=jnp.float32).astype(BF)
        sD = jnp.where(tri, sD, jnp.bfloat16(NEG_INF))
        mD = jnp.max(sD, axis=1, keepdims=True)
        if i > 0:
            sL = jax.lax.dot_general(qi, k[:lo], (((1,), (1,)), ((), ())),
                                     preferred_element_type=jnp.float32).astype(BF)
            m = jnp.maximum(mD, jnp.max(sL, axis=1, keepdims=True))
            pL = jnp.exp2(sL - m)
            pD = jnp.exp2(sD - m)
            acc = jnp.dot(pL, ve[:lo], preferred_element_type=jnp.float32)
            acc = acc + jnp.dot(pD, ve[lo:lo + TQ],
                                preferred_element_type=jnp.float32)
        else:
            pD = jnp.exp2(sD - mD)
            acc = jnp.dot(pD, ve[lo:lo + TQ],
                          preferred_element_type=jnp.float32)
        l = acc[:, DH:DH + 1]
        out_ref[hh, lo:lo + TQ, :] = (acc[:, :DH] / l).astype(BF)


def _mlp_kernel(a_ref, x_ref, wo_ref, bo_ref, wg_ref, bg_ref, wu_ref, bu_ref,
                wd_ref, bd_ref, lnw_ref, lnb_ref, out_ref):
    at = jnp.concatenate([a_ref[h] for h in range(NH)], axis=1)
    x2 = _dot_t(at, wo_ref[...]) + bo_ref[...] + x_ref[...]
    h = _ln(x2, lnw_ref[...], lnb_ref[...]).astype(BF)
    g = _dot_t(h, wg_ref[...]) + bg_ref[...]
    u = _dot_t(h, wu_ref[...]) + bu_ref[...]
    mlp = ((g * jax.nn.sigmoid(g)) * u).astype(BF)
    out_ref[...] = _dot_t(mlp, wd_ref[...]) + bd_ref[...] + x2


def kernel(x, Wq, bq, Wk, bk, Wv, bv, Wo, bo, Wg, bg, Wu, bu, Wd, bd,
           ln1_w, ln1_b, ln2_w, ln2_b):
    xs = x.reshape(S, H)
    bqkv = jnp.concatenate([bq, bk, bv]).reshape(1, 3 * H)

    full = lambda shape: pl.BlockSpec(shape, lambda i: (0,) * len(shape))

    qkv = pl.pallas_call(
        _qkv_kernel,
        grid=(S // TS,),
        in_specs=[
            pl.BlockSpec((TS, H), lambda i: (i, 0)),
            full((H, H)), full((H, H)), full((H, H)),
            full((1, 3 * H)), full((1, H)), full((1, H)),
        ],
        out_specs=pl.BlockSpec((TS, 3 * H), lambda i: (i, 0)),
        out_shape=jax.ShapeDtypeStruct((S, 3 * H), BF),
    )(xs, Wq.astype(BF), Wk.astype(BF), Wv.astype(BF), bqkv,
      ln1_w.reshape(1, H), ln1_b.reshape(1, H))

    qkv_h = qkv.reshape(S, 3 * NH, DH).transpose(1, 0, 2)

    attn = pl.pallas_call(
        _attn_kernel,
        grid=(NH // HPP,),
        in_specs=[
            pl.BlockSpec((HPP, S, DH), lambda h: (h, 0, 0)),
            pl.BlockSpec((HPP, S, DH), lambda h: (NH // HPP + h, 0, 0)),
            pl.BlockSpec((HPP, S, DH), lambda h: (2 * NH // HPP + h, 0, 0)),
        ],
        out_specs=pl.BlockSpec((HPP, S, DH), lambda h: (h, 0, 0)),
        out_shape=jax.ShapeDtypeStruct((NH, S, DH), BF),
    )(qkv_h, qkv_h, qkv_h)

    out = pl.pallas_call(
        _mlp_kernel,
        grid=(S // TS,),
        in_specs=[
            pl.BlockSpec((NH, TS, DH), lambda i: (0, i, 0)),
            pl.BlockSpec((TS, H), lambda i: (i, 0)),
            full((H, H)), full((1, H)),
            full((H, H)), full((1, H)),
            full((H, H)), full((1, H)),
            full((H, H)), full((1, H)),
            full((1, H)), full((1, H)),
        ],
        out_specs=pl.BlockSpec((TS, H), lambda i: (i, 0)),
        out_shape=jax.ShapeDtypeStruct((S, H), jnp.float32),
    )(attn, xs, Wo.astype(BF), bo.reshape(1, H), Wg.astype(BF), bg.reshape(1, H),
      Wu.astype(BF), bu.reshape(1, H), Wd.astype(BF), bd.reshape(1, H),
      ln2_w.reshape(1, H), ln2_b.reshape(1, H))

    return out.reshape(B, S, H)


# speedup vs baseline: 1.4291x; 1.2798x over previous
import jax
import jax.numpy as jnp
import numpy as np
from jax.experimental import pallas as pl

B, S, H, NH = 1, 2048, 1024, 16
DH = H // NH

TS = 256
TQ = 512

NEG_INF = -1e30
BF = jnp.bfloat16


def _ln(t, w, b, eps=1e-6):
    m = jnp.mean(t, axis=-1, keepdims=True)
    v = jnp.mean((t - m) ** 2, axis=-1, keepdims=True)
    return (t - m) * jax.lax.rsqrt(v + eps) * w + b


def _dot_t(a, w):
    return jax.lax.dot_general(a, w, (((1,), (1,)), ((), ())),
                               preferred_element_type=jnp.float32)


def _qkv_kernel(x_ref, wq_ref, wk_ref, wv_ref, b_ref, lnw_ref, lnb_ref, out_ref):
    h = _ln(x_ref[...], lnw_ref[...], lnb_ref[...]).astype(BF)
    b = b_ref[...]
    out_ref[:, :H] = (_dot_t(h, wq_ref[...]) + b[:, :H]).astype(BF)
    out_ref[:, H:2 * H] = (_dot_t(h, wk_ref[...]) + b[:, H:2 * H]).astype(BF)
    out_ref[:, 2 * H:] = (_dot_t(h, wv_ref[...]) + b[:, 2 * H:]).astype(BF)


def _attn_kernel(qkv_ref, out_ref):
    qscale = jnp.float32(np.log2(np.e) / np.sqrt(DH))
    tri = jax.lax.broadcasted_iota(jnp.int32, (TQ, TQ), 0) >= \
        jax.lax.broadcasted_iota(jnp.int32, (TQ, TQ), 1)

    for h, i in [(a, b) for b in range(S // TQ) for a in range(NH)]:
        k = qkv_ref[:, H + h * DH:H + (h + 1) * DH]
        ve = jnp.concatenate(
            [qkv_ref[:, 2 * H + h * DH:2 * H + (h + 1) * DH],
             jnp.ones((S, DH), BF)], axis=1)
        lo = i * TQ
        qi = (qkv_ref[lo:lo + TQ, h * DH:(h + 1) * DH].astype(jnp.float32)
              * qscale).astype(BF)
        sD = jax.lax.dot_general(qi, k[lo:lo + TQ], (((1,), (1,)), ((), ())),
                                 preferred_element_type=jnp.float32).astype(BF)
        sD = jnp.where(tri, sD, jnp.bfloat16(NEG_INF))
        mD = jnp.max(sD, axis=1, keepdims=True)
        if i > 0:
            sL = jax.lax.dot_general(qi, k[:lo], (((1,), (1,)), ((), ())),
                                     preferred_element_type=jnp.float32).astype(BF)
            m = jnp.maximum(mD, jnp.max(sL, axis=1, keepdims=True))
            pL = jnp.exp2(sL - m)
            pD = jnp.exp2(sD - m)
            acc = jnp.dot(pL, ve[:lo], preferred_element_type=jnp.float32)
            acc = acc + jnp.dot(pD, ve[lo:lo + TQ],
                                preferred_element_type=jnp.float32)
        else:
            pD = jnp.exp2(sD - mD)
            acc = jnp.dot(pD, ve[lo:lo + TQ],
                          preferred_element_type=jnp.float32)
        l = acc[:, DH:DH + 1]
        out_ref[lo:lo + TQ, h * DH:(h + 1) * DH] = (acc[:, :DH] / l).astype(BF)


def _mlp_kernel(a_ref, x_ref, wo_ref, bo_ref, wg_ref, bg_ref, wu_ref, bu_ref,
                wd_ref, bd_ref, lnw_ref, lnb_ref, out_ref):
    x2 = _dot_t(a_ref[...], wo_ref[...]) + bo_ref[...] + x_ref[...]
    h = _ln(x2, lnw_ref[...], lnb_ref[...]).astype(BF)
    g = _dot_t(h, wg_ref[...]) + bg_ref[...]
    u = _dot_t(h, wu_ref[...]) + bu_ref[...]
    mlp = ((g * jax.nn.sigmoid(g)) * u).astype(BF)
    out_ref[...] = _dot_t(mlp, wd_ref[...]) + bd_ref[...] + x2


def kernel(x, Wq, bq, Wk, bk, Wv, bv, Wo, bo, Wg, bg, Wu, bu, Wd, bd,
           ln1_w, ln1_b, ln2_w, ln2_b):
    xs = x.reshape(S, H)
    bqkv = jnp.concatenate([bq, bk, bv]).reshape(1, 3 * H)

    full = lambda shape: pl.BlockSpec(shape, lambda i: (0,) * len(shape))

    qkv = pl.pallas_call(
        _qkv_kernel,
        grid=(S // TS,),
        in_specs=[
            pl.BlockSpec((TS, H), lambda i: (i, 0)),
            full((H, H)), full((H, H)), full((H, H)),
            full((1, 3 * H)), full((1, H)), full((1, H)),
        ],
        out_specs=pl.BlockSpec((TS, 3 * H), lambda i: (i, 0)),
        out_shape=jax.ShapeDtypeStruct((S, 3 * H), BF),
    )(xs, Wq.astype(BF), Wk.astype(BF), Wv.astype(BF), bqkv,
      ln1_w.reshape(1, H), ln1_b.reshape(1, H))

    attn = pl.pallas_call(
        _attn_kernel,
        grid=(1,),
        in_specs=[full((S, 3 * H))],
        out_specs=full((S, H)),
        out_shape=jax.ShapeDtypeStruct((S, H), BF),
    )(qkv)

    out = pl.pallas_call(
        _mlp_kernel,
        grid=(S // TS,),
        in_specs=[
            pl.BlockSpec((TS, H), lambda i: (i, 0)),
            pl.BlockSpec((TS, H), lambda i: (i, 0)),
            full((H, H)), full((1, H)),
            full((H, H)), full((1, H)),
            full((H, H)), full((1, H)),
            full((H, H)), full((1, H)),
            full((1, H)), full((1, H)),
        ],
        out_specs=pl.BlockSpec((TS, H), lambda i: (i, 0)),
        out_shape=jax.ShapeDtypeStruct((S, H), jnp.float32),
    )(attn, xs, Wo.astype(BF), bo.reshape(1, H), Wg.astype(BF), bg.reshape(1, H),
      Wu.astype(BF), bu.reshape(1, H), Wd.astype(BF), bd.reshape(1, H),
      ln2_w.reshape(1, H), ln2_b.reshape(1, H))

    return out.reshape(B, S, H)
